# baseline (device time: 37534 ns/iter reference)
import jax
import jax.numpy as jnp
from jax import lax
from jax.experimental import pallas as pl
from jax.experimental.pallas import tpu as pltpu

N_DEV = 4
N_LAYERS = 3
DELTA = (0, 3, 1, 2)
S1R_WAIT = (1, 3, 0, 2)
S1L_WAIT = (2, 0, 3, 1)
PART_PREV = (3, 2, 1, 0)


def kernel(x, Win0, Wout0, Win1, Wout1, Win2, Wout2):
    m_per, d = x.shape
    M = N_DEV * m_per

    def body(x_ref, win0_ref, wout0_ref, win1_ref, wout1_ref, win2_ref,
             wout2_ref, out_ref, xg, psend, p_l, p_r, p_d, send_sems,
             recv_sems):
        my = lax.axis_index("i")
        left = jnp.mod(my - 1, N_DEV)
        right = jnp.mod(my + 1, N_DEV)
        diag = jnp.mod(my + 2, N_DEV)

        def mrc(src, dst, sem_idx, target):
            return pltpu.make_async_remote_copy(
                src_ref=src, dst_ref=dst,
                send_sem=send_sems.at[sem_idx],
                recv_sem=recv_sems.at[sem_idx],
                device_id=(target,),
                device_id_type=pl.DeviceIdType.MESH,
            )

        def slot(r):
            return slice(r * m_per, (r + 1) * m_per)

        barrier_sem = pltpu.get_barrier_semaphore()
        for nbr in (left, right):
            pl.semaphore_signal(
                barrier_sem, inc=1,
                device_id=(nbr,), device_id_type=pl.DeviceIdType.MESH,
            )

        weights = ((win0_ref, wout0_ref), (win1_ref, wout1_ref),
                   (win2_ref, wout2_ref))

        def gemm(li, xc):
            win_ref, wout_ref = weights[li]
            hid = jnp.maximum(
                lax.dot(xc, win_ref[:, :],
                        preferred_element_type=jnp.float32), 0.0)
            return lax.dot(hid, wout_ref[:, :],
                           preferred_element_type=jnp.float32)

        def push_block(li, r, part):
            b = 3 + 12 * li + 3 * r
            ps = li % 2
            psend[ps, slot(r), :] = part.astype(jnp.bfloat16)
            s1l = mrc(psend.at[ps, slot(r), :], p_r.at[li, slot(r), :],
                      b + 0, left)
            s1r = mrc(psend.at[ps, slot(r), :], p_l.at[li, slot(r), :],
                      b + 1, right)
            s1d = mrc(psend.at[ps, slot(r), :], p_d.at[li, slot(r), :],
                      b + 2, diag)
            s1l.start()
            s1r.start()
            s1d.start()
            return s1l, s1r, s1d

        def finish_rank(li_prev, parts, pend, r):
            pend[S1R_WAIT[r]][1].wait()
            pend[S1L_WAIT[r]][0].wait()
            pend[r][2].wait()
            remote = (
                (p_l[li_prev, slot(S1R_WAIT[r]), :]
                 + p_r[li_prev, slot(S1L_WAIT[r]), :])
                + p_d[li_prev, slot(r), :]
            )
            return parts[PART_PREV[r]] + remote.astype(jnp.float32)

        parts = [None] * N_DEV
        pend = [None] * N_DEV
        parts[0] = gemm(0, x_ref[:, :])
        pl.semaphore_wait(barrier_sem, 2)
        g_l = mrc(x_ref, xg.at[slot(2), :], 0, left)
        g_r = mrc(x_ref, xg.at[slot(1), :], 1, right)
        g_d = mrc(x_ref, xg.at[slot(3), :], 2, diag)
        g_l.start()
        g_r.start()
        g_d.start()
        pend[0] = push_block(0, 0, parts[0])
        g_r.wait()
        parts[1] = gemm(0, xg[slot(1), :])
        pend[1] = push_block(0, 1, parts[1])
        g_l.wait()
        parts[2] = gemm(0, xg[slot(2), :])
        pend[2] = push_block(0, 2, parts[2])
        g_d.wait()
        parts[3] = gemm(0, xg[slot(3), :])
        pend[3] = push_block(0, 3, parts[3])

        for li in range(1, N_LAYERS):
            nparts = [None] * N_DEV
            npend = [None] * N_DEV
            for r in range(N_DEV):
                xc = finish_rank(li - 1, parts, pend, r)
                nparts[r] = gemm(li, xc)
                npend[r] = push_block(li, r, nparts[r])
            parts, pend = nparts, npend

        for r in range(N_DEV):
            block = jnp.mod(my + 2 * N_LAYERS + DELTA[r], N_DEV)
            out_ref[pl.ds(block * m_per, m_per), :] = finish_rank(
                N_LAYERS - 1, parts, pend, r)

    n_sems = 3 + N_LAYERS * N_DEV * 3
    return pl.pallas_call(
        body,
        out_shape=jax.ShapeDtypeStruct((M, d), jnp.float32),
        in_specs=[pl.BlockSpec(memory_space=pltpu.VMEM)] * 7,
        out_specs=pl.BlockSpec(memory_space=pltpu.VMEM),
        scratch_shapes=[
            pltpu.VMEM((M, d), jnp.float32),
            pltpu.VMEM((2, M, d), jnp.bfloat16),
            pltpu.VMEM((N_LAYERS, M, d), jnp.bfloat16),
            pltpu.VMEM((N_LAYERS, M, d), jnp.bfloat16),
            pltpu.VMEM((N_LAYERS, M, d), jnp.bfloat16),
            pltpu.SemaphoreType.DMA((n_sems,)),
            pltpu.SemaphoreType.DMA((n_sems,)),
        ],
        compiler_params=pltpu.CompilerParams(collective_id=0),
    )(x, Win0, Wout0, Win1, Wout1, Win2, Wout2)
